# baseline (speedup 1.0000x reference)
import jax
import jax.numpy as jnp
from jax import lax
from jax.experimental import pallas as pl
from jax.experimental.pallas import tpu as pltpu
from jax.experimental.pallas import tpu_sc as plsc

B = 16384
D = 64
NW = 32
BW = B // NW
NCHUNK = 4
CHUNK = BW // NCHUNK
GROUPS = BW // 16
MAGIC = 0x5F3759DF


def _sqrt_pos(x):
    y = lax.bitcast_convert_type(
        jnp.int32(MAGIC) - (lax.bitcast_convert_type(x, jnp.int32) >> 1),
        jnp.float32)
    half = x * 0.5
    for _ in range(3):
        y = y * (1.5 - half * y * y)
    return x * y


def _body(uidx_hbm, iidx_hbm, tt_hbm, out_hbm,
          uidx_v, iidx_v, a_v, b_v, out_v, sem_a, sem_b):
    wid = lax.axis_index("s") * 2 + lax.axis_index("c")

    pltpu.sync_copy(uidx_hbm.at[wid], uidx_v)
    pltpu.sync_copy(iidx_hbm.at[wid], iidx_v)

    def fetch(d, _):
        for j in range(NCHUNK):
            dst = pl.ds(j * CHUNK, CHUNK)
            pltpu.async_copy(tt_hbm.at[d].at[uidx_v.at[j]],
                             a_v.at[d].at[dst], sem_a)
            pltpu.async_copy(tt_hbm.at[d].at[iidx_v.at[j]],
                             b_v.at[d].at[dst], sem_b)
        return 0

    lax.fori_loop(0, D, fetch, 0)
    pltpu.make_async_copy(tt_hbm.at[pl.ds(0, D), pl.ds(0, BW)], a_v,
                          sem_a).wait()
    pltpu.make_async_copy(tt_hbm.at[pl.ds(0, D), pl.ds(0, BW)], b_v,
                          sem_b).wait()

    zero = jnp.zeros((16,), jnp.float32)

    for g in range(GROUPS):
        cols = pl.ds(g * 16, 16)

        def dstep(i, carry):
            sn, sa, sb = carry
            d0 = i * 4
            for u in range(4):
                av = a_v[d0 + u, cols]
                bv = b_v[d0 + u, cols]
                sn = sn + av * bv
                sa = sa + av * av
                sb = sb + bv * bv
            return sn, sa, sb

        sn, sa, sb = lax.fori_loop(0, D // 4, dstep, (zero, zero, zero))

        na = jnp.maximum(_sqrt_pos(sa), 1e-8)
        nb = jnp.maximum(_sqrt_pos(sb), 1e-8)
        out_v[pl.ds(g * 16, 16)] = sn / (na * nb)

    pltpu.sync_copy(out_v, out_hbm.at[pl.ds(wid * BW, BW)])


def kernel(user_idx, item_idx, user_table, item_table):
    del item_table
    uidx = user_idx.astype(jnp.int32).reshape(NW, NCHUNK, CHUNK)
    iidx = item_idx.astype(jnp.int32).reshape(NW, NCHUNK, CHUNK)
    tt = user_table.T

    f = pl.kernel(
        _body,
        out_type=jax.ShapeDtypeStruct((B,), jnp.float32),
        mesh=plsc.VectorSubcoreMesh(core_axis_name="c", subcore_axis_name="s"),
        compiler_params=pltpu.CompilerParams(
            needs_layout_passes=False, use_tc_tiling_on_sc=False),
        scratch_types=[
            pltpu.VMEM((NCHUNK, CHUNK), jnp.int32),
            pltpu.VMEM((NCHUNK, CHUNK), jnp.int32),
            pltpu.VMEM((D, BW), jnp.float32),
            pltpu.VMEM((D, BW), jnp.float32),
            pltpu.VMEM((BW,), jnp.float32),
            pltpu.SemaphoreType.DMA,
            pltpu.SemaphoreType.DMA,
        ],
    )
    out = f(uidx, iidx, tt)
    return out.reshape(B, 1)


# speedup vs baseline: 3.5427x; 3.5427x over previous
import jax
import jax.numpy as jnp
from jax import lax
from jax.experimental import pallas as pl
from jax.experimental.pallas import tpu as pltpu
from jax.experimental.pallas import tpu_sc as plsc

B = 16384
D = 64
NW = 32
BW = B // NW
NCHUNK = 4
CHUNK = BW // NCHUNK
GROUPS = BW // 16
HALF_D = D // 2
MAGIC = 0x5F3759DF


def _sqrt_pos(x):
    y = lax.bitcast_convert_type(
        jnp.int32(MAGIC) - (lax.bitcast_convert_type(x, jnp.int32) >> 1),
        jnp.float32)
    half = x * 0.5
    for _ in range(3):
        y = y * (1.5 - half * y * y)
    return x * y


def _body(uidx_hbm, iidx_hbm, t0_hbm, t1_hbm, out_hbm,
          uidx_v, iidx_v, a0_v, a1_v, b0_v, b1_v, out_v, sem):
    wid = lax.axis_index("s") * 2 + lax.axis_index("c")

    pltpu.sync_copy(uidx_hbm.at[wid], uidx_v)
    pltpu.sync_copy(iidx_hbm.at[wid], iidx_v)

    copies = []
    for j in range(NCHUNK):
        rows = pl.ds(j * CHUNK, CHUNK)
        idx_u = uidx_v.at[j]
        idx_i = iidx_v.at[j]
        copies.append(pltpu.async_copy(t0_hbm.at[idx_u], a0_v.at[rows], sem))
        copies.append(pltpu.async_copy(t1_hbm.at[idx_u], a1_v.at[rows], sem))
        copies.append(pltpu.async_copy(t0_hbm.at[idx_i], b0_v.at[rows], sem))
        copies.append(pltpu.async_copy(t1_hbm.at[idx_i], b1_v.at[rows], sem))
    for c in copies:
        c.wait()

    lane = lax.iota(jnp.int32, 16)
    zero = jnp.zeros((16,), jnp.float32)

    for g in range(GROUPS):
        row_ids = lane + (g * 16)

        def dstep(i, carry):
            sn, sa, sb = carry
            d0 = i * 4
            for u in range(4):
                col = jnp.full((16,), d0 + u, jnp.int32)
                av = plsc.load_gather(a0_v, [row_ids, col])
                bv = plsc.load_gather(b0_v, [row_ids, col])
                sn = sn + av * bv
                sa = sa + av * av
                sb = sb + bv * bv
                av = plsc.load_gather(a1_v, [row_ids, col])
                bv = plsc.load_gather(b1_v, [row_ids, col])
                sn = sn + av * bv
                sa = sa + av * av
                sb = sb + bv * bv
            return sn, sa, sb

        sn, sa, sb = lax.fori_loop(0, HALF_D // 4, dstep, (zero, zero, zero))

        na = jnp.maximum(_sqrt_pos(sa), 1e-8)
        nb = jnp.maximum(_sqrt_pos(sb), 1e-8)
        out_v[pl.ds(g * 16, 16)] = sn / (na * nb)

    pltpu.sync_copy(out_v, out_hbm.at[wid])


def kernel(user_idx, item_idx, user_table, item_table):
    del item_table
    uidx = user_idx.astype(jnp.int32).reshape(NW, NCHUNK, CHUNK)
    iidx = item_idx.astype(jnp.int32).reshape(NW, NCHUNK, CHUNK)
    t0 = user_table[:, :HALF_D]
    t1 = user_table[:, HALF_D:]

    f = pl.kernel(
        _body,
        out_type=jax.ShapeDtypeStruct((NW, BW), jnp.float32),
        mesh=plsc.VectorSubcoreMesh(core_axis_name="c", subcore_axis_name="s"),
        compiler_params=pltpu.CompilerParams(
            needs_layout_passes=False, use_tc_tiling_on_sc=False),
        scratch_types=[
            pltpu.VMEM((NCHUNK, CHUNK), jnp.int32),
            pltpu.VMEM((NCHUNK, CHUNK), jnp.int32),
            pltpu.VMEM((BW, HALF_D), jnp.float32),
            pltpu.VMEM((BW, HALF_D), jnp.float32),
            pltpu.VMEM((BW, HALF_D), jnp.float32),
            pltpu.VMEM((BW, HALF_D), jnp.float32),
            pltpu.VMEM((BW,), jnp.float32),
            pltpu.SemaphoreType.DMA,
        ],
    )
    out = f(uidx, iidx, t0, t1)
    return out.reshape(B, 1)


# speedup vs baseline: 8.5904x; 2.4248x over previous
import jax
import jax.numpy as jnp
from jax import lax
from jax.experimental import pallas as pl
from jax.experimental.pallas import tpu as pltpu
from jax.experimental.pallas import tpu_sc as plsc

B = 16384
D = 64
DP = 128
NW = 32
BW = B // NW
PASS_ROWS = 256
NCHUNK = 2
CHUNK = 128
PASS_GROUPS = PASS_ROWS // 16
MAGIC = 0x5F3759DF


def _sqrt_pos(x):
    y = lax.bitcast_convert_type(
        jnp.int32(MAGIC) - (lax.bitcast_convert_type(x, jnp.int32) >> 1),
        jnp.float32)
    half = x * 0.5
    for _ in range(3):
        y = y * (1.5 - half * y * y)
    return x * y


def _body(uidx_hbm, iidx_hbm, table_hbm, out_hbm,
          uidx_v, iidx_v, a_v, b_v, out_v, sem):
    wid = lax.axis_index("s") * 2 + lax.axis_index("c")

    lane = lax.iota(jnp.int32, 16)
    zero = jnp.zeros((16,), jnp.float32)

    for p in range(2):
        pltpu.sync_copy(uidx_hbm.at[wid, pl.ds(p * NCHUNK, NCHUNK)], uidx_v)
        pltpu.sync_copy(iidx_hbm.at[wid, pl.ds(p * NCHUNK, NCHUNK)], iidx_v)

        copies = []
        for j in range(NCHUNK):
            rows = pl.ds(j * CHUNK, CHUNK)
            copies.append(pltpu.async_copy(
                table_hbm.at[uidx_v.at[j]], a_v.at[rows], sem))
            copies.append(pltpu.async_copy(
                table_hbm.at[iidx_v.at[j]], b_v.at[rows], sem))
        for c in copies:
            c.wait()

        for g in range(PASS_GROUPS):
            row_ids = lane + (g * 16)

            def dstep(i, carry):
                sn, sa, sb = carry
                d0 = i * 4
                for u in range(4):
                    col = jnp.full((16,), d0 + u, jnp.int32)
                    av = plsc.load_gather(a_v, [row_ids, col])
                    bv = plsc.load_gather(b_v, [row_ids, col])
                    sn = sn + av * bv
                    sa = sa + av * av
                    sb = sb + bv * bv
                return sn, sa, sb

            sn, sa, sb = lax.fori_loop(0, D // 4, dstep, (zero, zero, zero))

            na = jnp.maximum(_sqrt_pos(sa), 1e-8)
            nb = jnp.maximum(_sqrt_pos(sb), 1e-8)
            slot = p * PASS_GROUPS + g
            out_v[slot // 8, pl.ds((slot % 8) * 16, 16)] = sn / (na * nb)

    pltpu.sync_copy(out_v, out_hbm.at[pl.ds(wid * 4, 4)])


def kernel(user_idx, item_idx, user_table, item_table):
    del item_table
    uidx = user_idx.astype(jnp.int32).reshape(NW, 2 * NCHUNK, CHUNK)
    iidx = item_idx.astype(jnp.int32).reshape(NW, 2 * NCHUNK, CHUNK)
    tp = jnp.pad(user_table, ((0, 0), (0, DP - D)))

    f = pl.kernel(
        _body,
        out_type=jax.ShapeDtypeStruct((NW * 4, 128), jnp.float32),
        mesh=plsc.VectorSubcoreMesh(core_axis_name="c", subcore_axis_name="s"),
        compiler_params=pltpu.CompilerParams(
            needs_layout_passes=False, use_tc_tiling_on_sc=True),
        scratch_types=[
            pltpu.VMEM((NCHUNK, CHUNK), jnp.int32),
            pltpu.VMEM((NCHUNK, CHUNK), jnp.int32),
            pltpu.VMEM((PASS_ROWS, DP), jnp.float32),
            pltpu.VMEM((PASS_ROWS, DP), jnp.float32),
            pltpu.VMEM((4, 128), jnp.float32),
            pltpu.SemaphoreType.DMA,
        ],
    )
    out = f(uidx, iidx, tp)
    return out.reshape(B, 1)
